# Initial kernel scaffold; baseline (speedup 1.0000x reference)
import jax
import jax.numpy as jnp
from jax.experimental import pallas as pl


def kernel(p, params):
    raise NotImplementedError("write your pallas kernel here")


# speedup vs baseline: 3.5222x; 3.5222x over previous
import functools
import math

import jax
import jax.numpy as jnp
from jax import lax
from jax.experimental import pallas as pl
from jax.experimental.pallas import tpu as pltpu

_N_POINTS = 20000
_MAX_K = 64
_SA1_RATIO = 0.1
_SA1_R = 0.2
_SA2_RATIO = 0.05
_SA2_R = 0.4


def _fps_body(px_ref, py_ref, pz_ref, out_ref, dists_ref, *, n_samples, n_valid):
    R, C = px_ref.shape
    px = px_ref[...]
    py = py_ref[...]
    pz = pz_ref[...]
    flat = (lax.broadcasted_iota(jnp.int32, (R, C), 0) * C
            + lax.broadcasted_iota(jnp.int32, (R, C), 1))
    validm = flat < n_valid
    x0 = px_ref[0, 0]
    y0 = py_ref[0, 0]
    z0 = pz_ref[0, 0]
    d0 = (px - x0) ** 2 + (py - y0) ** 2 + (pz - z0) ** 2
    dists_ref[...] = jnp.where(validm, d0, -jnp.inf)

    SR, SC_ = out_ref.shape
    slot = (lax.broadcasted_iota(jnp.int32, (SR, SC_), 0) * SC_
            + lax.broadcasted_iota(jnp.int32, (SR, SC_), 1))
    idxbuf0 = jnp.zeros((SR, SC_), jnp.int32)

    def body(i, idxbuf):
        dists = dists_ref[...]
        m = jnp.max(dists)
        nxt = jnp.min(jnp.where(dists == m, flat, jnp.int32(2 ** 30)))
        sel = flat == nxt
        cx = jnp.sum(jnp.where(sel, px, 0.0))
        cy = jnp.sum(jnp.where(sel, py, 0.0))
        cz = jnp.sum(jnp.where(sel, pz, 0.0))
        d = (px - cx) ** 2 + (py - cy) ** 2 + (pz - cz) ** 2
        dists_ref[...] = jnp.where(validm, jnp.minimum(dists, d), -jnp.inf)
        return jnp.where(slot == i, nxt, idxbuf)

    idxbuf = lax.fori_loop(1, n_samples, body, idxbuf0, unroll=False)
    out_ref[...] = idxbuf


def _fps(pos, n_samples):
    n = pos.shape[0]
    rows = -(-n // 128)
    npad = rows * 128
    pcols = jnp.pad(pos, ((0, npad - n), (0, 0)))
    px = pcols[:, 0].reshape(rows, 128)
    py = pcols[:, 1].reshape(rows, 128)
    pz = pcols[:, 2].reshape(rows, 128)
    srows = -(-n_samples // 128)
    out = pl.pallas_call(
        functools.partial(_fps_body, n_samples=n_samples, n_valid=n),
        out_shape=jax.ShapeDtypeStruct((srows, 128), jnp.int32),
        scratch_shapes=[pltpu.VMEM((rows, 128), jnp.float32)],
    )(px, py, pz)
    return out.reshape(-1)[:n_samples]


def _mlp_apply(layers, x):
    eps = 1e-05
    for layer in layers:
        x = x @ layer['W'] + layer['b']
        x = jax.nn.relu(x)
        x = layer['gamma'] * x / jnp.sqrt(1.0 + eps) + layer['beta']
    return x


def _radius(pos_all, centers, r, max_k):
    d2 = (jnp.sum(centers ** 2, axis=1)[:, None]
          + jnp.sum(pos_all ** 2, axis=1)[None, :]
          - 2.0 * (centers @ pos_all.T))
    neg = jnp.where(d2 <= r * r, -d2, -jnp.inf)
    vals, idx = jax.lax.top_k(neg, max_k)
    valid = vals > -jnp.inf
    return idx, valid


def _sa_module(layers, x, pos, ratio, r):
    n = int(math.ceil(ratio * pos.shape[0]))
    idx = _fps(pos, n)
    centers = pos[idx]
    nbr, valid = _radius(pos, centers, r, _MAX_K)
    x_j = x[nbr]
    rel = pos[nbr] - centers[:, None, :]
    msg = jnp.concatenate([x_j, rel], axis=-1)
    h = _mlp_apply(layers, msg.reshape(-1, msg.shape[-1])).reshape(n, _MAX_K, -1)
    h = jnp.where(valid[:, :, None], h, -jnp.inf)
    out = jnp.max(h, axis=1)
    out = jnp.where(jnp.isfinite(out), out, 0.0)
    return out, centers


def kernel(p, params):
    x0 = p
    b0 = jnp.zeros((p.shape[0],), jnp.int32)
    x1, p1 = _sa_module(params['sa1'], x0, p, _SA1_RATIO, _SA1_R)
    x2, p2 = _sa_module(params['sa2'], x1, p1, _SA2_RATIO, _SA2_R)
    h3 = _mlp_apply(params['sa3'], jnp.concatenate([x2, p2], axis=1))
    x3 = jnp.max(h3, axis=0, keepdims=True)
    p3 = jnp.zeros((1, 3), jnp.float32)
    b1 = jnp.zeros((p1.shape[0],), jnp.int32)
    b2 = jnp.zeros((p2.shape[0],), jnp.int32)
    b3 = jnp.arange(1, dtype=jnp.int32)
    return (p, p1, p2, p3, x0, x1, x2, x3, b0, b1, b2, b3)
